# baseline (speedup 1.0000x reference)
import functools

import jax
import jax.numpy as jnp
from jax import lax
from jax.experimental import pallas as pl
from jax.experimental.pallas import tpu as pltpu
from jax.experimental.pallas import tpu_sc as plsc

_D = 1024
_B = 4
_S = 2048
_NW = 32
_SPW = _S // _NW
_ROWS = 8
_CPB = _SPW // _ROWS
_NCH = _B * _CPB
_NVEC = _D // 16


def _sc_body(x_hbm, codes_hbm, w_hbm, abs_hbm, out_hbm,
             abs_v, in0, in1, ou0, ou1, w_v, codes_v,
             isem0, isem1, osem0, osem1):
    wid = lax.axis_index("s") * 2 + lax.axis_index("c")
    s_base = wid * _SPW
    ins = (in0, in1)
    ous = (ou0, ou1)
    isems = (isem0, isem1)
    osems = (osem0, osem1)

    pltpu.sync_copy(w_hbm, w_v)
    pltpu.sync_copy(abs_hbm.at[pl.ds(s_base, _SPW)], abs_v)
    for bb in range(_B):
        pltpu.sync_copy(codes_hbm.at[pl.ds(bb * _S + s_base, _SPW)],
                        codes_v.at[pl.ds(bb * _SPW, _SPW)])

    def x_base(m):
        return (m // _CPB) * _S + s_base + (m % _CPB) * _ROWS

    for b in range(2):
        pltpu.async_copy(x_hbm.at[pl.ds(x_base(b), _ROWS)], ins[b], isems[b])

    def step(k, _):
        for b in range(2):
            m = 2 * k + b
            pltpu.make_async_copy(
                x_hbm.at[pl.ds(0, _ROWS)], ins[b], isems[b]).wait()
            @pl.when(k > 0)
            def _():
                pltpu.make_async_copy(
                    ous[b], out_hbm.at[pl.ds(0, _ROWS)], osems[b]).wait()

            coff = (m // _CPB) * _SPW + (m % _CPB) * _ROWS
            aoff = (m % _CPB) * _ROWS

            @plsc.parallel_loop(0, _ROWS, 1, unroll=2)
            def _(t, b=b, coff=coff, aoff=aoff):
                cvec = codes_v[pl.ds(coff + t, 16)]
                c = cvec[0]

                @pl.when(c == 0)
                def _():
                    for g in range(_NVEC // 8):
                        sls = [pl.ds((8 * g + q) * 16, 16) for q in range(8)]
                        xs = [ins[b][t, sl] for sl in sls]
                        avs = [abs_v[aoff + t, sl] for sl in sls]
                        for sl, xv, av in zip(sls, xs, avs):
                            ous[b][t, sl] = xv + av

                @pl.when(c != 0)
                def _():
                    for g in range(_NVEC // 8):
                        sls = [pl.ds((8 * g + q) * 16, 16) for q in range(8)]
                        xs = [ins[b][t, sl] for sl in sls]
                        avs = [abs_v[aoff + t, sl] for sl in sls]
                        wvs = [w_v[c, sl] for sl in sls]
                        tmps = [xv + av for xv, av in zip(xs, avs)]
                        for sl, tmp, wv in zip(sls, tmps, wvs):
                            ous[b][t, sl] = tmp + wv
            pltpu.async_copy(ous[b], out_hbm.at[pl.ds(x_base(m), _ROWS)],
                             osems[b])

            @pl.when(k < (_NCH // 2 - 1))
            def _():
                pltpu.async_copy(x_hbm.at[pl.ds(x_base(m + 2), _ROWS)],
                                 ins[b], isems[b])
        return 0

    lax.fori_loop(0, _NCH // 2, step, 0)
    for b in range(2):
        pltpu.make_async_copy(
            ous[b], out_hbm.at[pl.ds(0, _ROWS)], osems[b]).wait()


def kernel(x, pos_codes, struct_w, abs_emb):
    b, s, d = x.shape
    x2 = x.reshape(b * s, d)
    codes = pos_codes.astype(jnp.int32).reshape(b * s)
    mesh = plsc.VectorSubcoreMesh(core_axis_name="c", subcore_axis_name="s")
    run = functools.partial(
        pl.kernel,
        mesh=mesh,
        out_type=jax.ShapeDtypeStruct((b * s, d), jnp.float32),
        scratch_types=[
            pltpu.VMEM((_SPW, _D), jnp.float32),
            pltpu.VMEM((_ROWS, _D), jnp.float32),
            pltpu.VMEM((_ROWS, _D), jnp.float32),
            pltpu.VMEM((_ROWS, _D), jnp.float32),
            pltpu.VMEM((_ROWS, _D), jnp.float32),
            pltpu.VMEM((5, _D), jnp.float32),
            pltpu.VMEM((_B * _SPW + 16,), jnp.int32),
            pltpu.SemaphoreType.DMA,
            pltpu.SemaphoreType.DMA,
            pltpu.SemaphoreType.DMA,
            pltpu.SemaphoreType.DMA,
        ],
    )(_sc_body)
    out = run(x2, codes, struct_w, abs_emb)
    return out.reshape(b, s, d)


# speedup vs baseline: 1.0346x; 1.0346x over previous
import functools

import jax
import jax.numpy as jnp
from jax import lax
from jax.experimental import pallas as pl
from jax.experimental.pallas import tpu as pltpu
from jax.experimental.pallas import tpu_sc as plsc

_D = 1024
_B = 4
_S = 2048
_NW = 32
_SPW = _S // _NW
_ROWS = 8
_CPB = _SPW // _ROWS
_NCH = _B * _CPB
_NVEC = _D // 16


def _sc_body(x_hbm, codes_hbm, w_hbm, abs_hbm, out_hbm,
             abs_v, in0, in1, ou0, ou1, w_v, codes_v,
             isem0, isem1, osem0, osem1):
    wid = lax.axis_index("s") * 2 + lax.axis_index("c")
    s_base = wid * _SPW
    ins = (in0, in1)
    ous = (ou0, ou1)
    isems = (isem0, isem1)
    osems = (osem0, osem1)

    pltpu.sync_copy(w_hbm, w_v)
    pltpu.sync_copy(abs_hbm.at[pl.ds(s_base, _SPW)], abs_v)
    for bb in range(_B):
        pltpu.sync_copy(codes_hbm.at[pl.ds(bb * _S + s_base, _SPW)],
                        codes_v.at[pl.ds(bb * _SPW, _SPW)])

    def x_base(m):
        return (m // _CPB) * _S + s_base + (m % _CPB) * _ROWS

    for b in range(2):
        pltpu.async_copy(x_hbm.at[pl.ds(x_base(b), _ROWS)], ins[b], isems[b])

    def step(k, _):
        for b in range(2):
            m = 2 * k + b
            pltpu.make_async_copy(
                x_hbm.at[pl.ds(0, _ROWS)], ins[b], isems[b]).wait()
            @pl.when(k > 0)
            def _():
                pltpu.make_async_copy(
                    ous[b], out_hbm.at[pl.ds(0, _ROWS)], osems[b]).wait()

            coff = (m // _CPB) * _SPW + (m % _CPB) * _ROWS
            aoff = (m % _CPB) * _ROWS

            @plsc.parallel_loop(0, _ROWS, 1, unroll=2)
            def _(t, b=b, coff=coff, aoff=aoff):
                cvec = codes_v[pl.ds(coff + t, 16)]
                c = cvec[0]
                for g in range(_NVEC // 8):
                    sls = [pl.ds((8 * g + q) * 16, 16) for q in range(8)]
                    xs = [ins[b][t, sl] for sl in sls]
                    avs = [abs_v[aoff + t, sl] for sl in sls]
                    wvs = [w_v[c, sl] for sl in sls]
                    tmps = [xv + av for xv, av in zip(xs, avs)]
                    for sl, tmp, wv in zip(sls, tmps, wvs):
                        ous[b][t, sl] = tmp + wv
            pltpu.async_copy(ous[b], out_hbm.at[pl.ds(x_base(m), _ROWS)],
                             osems[b])

            @pl.when(k < (_NCH // 2 - 1))
            def _():
                pltpu.async_copy(x_hbm.at[pl.ds(x_base(m + 2), _ROWS)],
                                 ins[b], isems[b])
        return 0

    lax.fori_loop(0, _NCH // 2, step, 0)
    for b in range(2):
        pltpu.make_async_copy(
            ous[b], out_hbm.at[pl.ds(0, _ROWS)], osems[b]).wait()


def kernel(x, pos_codes, struct_w, abs_emb):
    b, s, d = x.shape
    x2 = x.reshape(b * s, d)
    codes = pos_codes.astype(jnp.int32).reshape(b * s)
    mesh = plsc.VectorSubcoreMesh(core_axis_name="c", subcore_axis_name="s")
    run = functools.partial(
        pl.kernel,
        mesh=mesh,
        out_type=jax.ShapeDtypeStruct((b * s, d), jnp.float32),
        scratch_types=[
            pltpu.VMEM((_SPW, _D), jnp.float32),
            pltpu.VMEM((_ROWS, _D), jnp.float32),
            pltpu.VMEM((_ROWS, _D), jnp.float32),
            pltpu.VMEM((_ROWS, _D), jnp.float32),
            pltpu.VMEM((_ROWS, _D), jnp.float32),
            pltpu.VMEM((5, _D), jnp.float32),
            pltpu.VMEM((_B * _SPW + 16,), jnp.int32),
            pltpu.SemaphoreType.DMA,
            pltpu.SemaphoreType.DMA,
            pltpu.SemaphoreType.DMA,
            pltpu.SemaphoreType.DMA,
        ],
    )(_sc_body)
    out = run(x2, codes, struct_w, abs_emb)
    return out.reshape(b, s, d)
